# baseline (speedup 1.0000x reference)
import jax
import jax.numpy as jnp
from jax.experimental import pallas as pl
from jax.experimental.pallas import tpu as pltpu

_H = 128
_W = 2 * _H
_ROWS = 2048
_CORES = 2


def _mlp_body(obs_ref, w1_ref, b1_ref, w2_ref, b2_ref, w3_ref, b3_ref, r_ref):
    x = obs_ref[...]
    h = jnp.dot(x, w1_ref[...], preferred_element_type=jnp.float32)
    h = jnp.maximum(h + b1_ref[...], 0.0)
    h = jnp.dot(h, w2_ref[...], preferred_element_type=jnp.float32)
    h = jnp.maximum(h + b2_ref[...], 0.0)
    o = jnp.dot(h, w3_ref[...], preferred_element_type=jnp.float32) + b3_ref[...]
    d = o[:, :_H] - o[:, _H:]
    sq = (d * d).reshape(_ROWS // 128, 128, 128)
    r_ref[...] = jnp.sum(sq, axis=-1) * (1.0 / _H)


def _norm_body(mean_ref, m2_ref, count_ref, r_ref, out_ref):
    r = r_ref[...]
    n = jnp.float32(r.size)
    bm = jnp.sum(r) / n
    dv = r - bm
    bm2 = jnp.sum(dv * dv)
    cnt = count_ref[0]
    new_count = cnt + n
    delta = bm - mean_ref[0]
    new_mean = mean_ref[0] + delta * n / new_count
    new_m2 = m2_ref[0] + bm2 + delta * delta * cnt * n / new_count
    std = jnp.where(new_count > 1.0, jnp.sqrt(new_m2 / (new_count - 1.0)), 1.0)
    out_ref[...] = (r - new_mean) * (1.0 / (std + 1e-8))


def kernel(obs, reward_mean, reward_m2, reward_count,
           tW1, tb1, tW2, tb2, tW3, tb3,
           pW1, pb1, pW2, pb2, pW3, pb3):
    batch, obs_dim = obs.shape
    z = jnp.zeros((_H, _H), jnp.float32)
    w1 = jnp.concatenate([tW1.T, pW1.T], axis=1)
    b1 = jnp.concatenate([tb1, pb1])[None, :]
    w2 = jnp.concatenate(
        [jnp.concatenate([tW2.T, z], axis=1),
         jnp.concatenate([z, pW2.T], axis=1)], axis=0)
    b2 = jnp.concatenate([tb2, pb2])[None, :]
    w3 = jnp.concatenate(
        [jnp.concatenate([tW3.T, z], axis=1),
         jnp.concatenate([z, pW3.T], axis=1)], axis=0)
    b3 = jnp.concatenate([tb3, pb3])[None, :]

    steps = batch // (_CORES * _ROWS)
    rrows = _ROWS // 128

    rewards = pl.pallas_call(
        _mlp_body,
        grid=(_CORES, steps),
        in_specs=[
            pl.BlockSpec((_ROWS, obs_dim), lambda c, i, s=steps: (c * s + i, 0)),
            pl.BlockSpec((obs_dim, _W), lambda c, i: (0, 0)),
            pl.BlockSpec((1, _W), lambda c, i: (0, 0)),
            pl.BlockSpec((_W, _W), lambda c, i: (0, 0)),
            pl.BlockSpec((1, _W), lambda c, i: (0, 0)),
            pl.BlockSpec((_W, _W), lambda c, i: (0, 0)),
            pl.BlockSpec((1, _W), lambda c, i: (0, 0)),
        ],
        out_specs=pl.BlockSpec((rrows, 128), lambda c, i, s=steps: (c * s + i, 0)),
        out_shape=jax.ShapeDtypeStruct((batch // 128, 128), jnp.float32),
        compiler_params=pltpu.CompilerParams(
            dimension_semantics=("parallel", "arbitrary"),
        ),
    )(obs, w1, b1, w2, b2, w3, b3)

    normalized = pl.pallas_call(
        _norm_body,
        in_specs=[
            pl.BlockSpec(memory_space=pltpu.SMEM),
            pl.BlockSpec(memory_space=pltpu.SMEM),
            pl.BlockSpec(memory_space=pltpu.SMEM),
            pl.BlockSpec((batch // 128, 128), lambda: (0, 0)),
        ],
        out_specs=pl.BlockSpec((batch // 128, 128), lambda: (0, 0)),
        out_shape=jax.ShapeDtypeStruct((batch // 128, 128), jnp.float32),
    )(reward_mean, reward_m2, reward_count, rewards)

    return normalized.reshape(batch)
